# Initial kernel scaffold; baseline (speedup 1.0000x reference)
import jax
import jax.numpy as jnp
from jax.experimental import pallas as pl


def kernel(fushed_features, input_data, adj_mx_topk_index, centroids, W1, b1, W2, b2, bn_weight, bn_bias):
    raise NotImplementedError("write your pallas kernel here")


# speedup vs baseline: 11.7510x; 11.7510x over previous
import jax
import jax.numpy as jnp
from jax import lax
from jax.experimental import pallas as pl
from jax.experimental.pallas import tpu as pltpu
from jax.experimental.pallas import tpu_sc as plsc

B, N, K, C, D = 2, 10000, 16, 8, 64
UPDATE_RATE = 0.01
MARGIN = 0.5

NC, NS = 2, 16
NW = NC * NS
TOT = B * N
PER_W = TOT // NW
CHUNK = 25
NCHUNK = PER_W // CHUNK

NB_A = 1000


def _stage_a_body(f_ref, adj_ref, cent_ref, w1_ref, b1_ref, w2_ref, b2_ref,
                  bnw_ref, bnb_ref, simi_ref, idx2_ref):
    f0 = f_ref[0]
    f1 = f_ref[1]
    inv_bd = 1.0 / (B * D)
    mean = (jnp.sum(f0, axis=1, keepdims=True)
            + jnp.sum(f1, axis=1, keepdims=True)) * inv_bd
    d0 = f0 - mean
    d1 = f1 - mean
    var = (jnp.sum(d0 * d0, axis=1, keepdims=True)
           + jnp.sum(d1 * d1, axis=1, keepdims=True)) * inv_bd
    scale = bnw_ref[...] * lax.rsqrt(var + 1e-5)
    bias = bnb_ref[...]

    w1a = w1_ref[0:D, :]
    w1b = w1_ref[D:2 * D, :]
    cpart = jnp.dot(cent_ref[...], w1b,
                    preferred_element_type=jnp.float32) + b1_ref[...]
    w2row = w2_ref[...]
    b2 = b2_ref[...]

    for b, db in ((0, d0), (1, d1)):
        ffn = db * scale + bias
        xp = jnp.dot(ffn, w1a, preferred_element_type=jnp.float32)
        cols = []
        for c in range(C):
            h = jnp.maximum(xp + cpart[c:c + 1, :], 0.0)
            sc = jnp.sum(h * w2row, axis=1, keepdims=True)
            cols.append(jnp.maximum(sc + b2, 0.0))
        s = jnp.concatenate(cols, axis=1)
        m = jnp.max(s, axis=1, keepdims=True)
        e = jnp.exp(s - m)
        simi = e / jnp.sum(e, axis=1, keepdims=True)
        pad = jnp.zeros((simi.shape[0], 16 - C), jnp.float32)
        simi_ref[b] = jnp.concatenate([simi, pad], axis=1)
        idx2_ref[b] = adj_ref[b] + b * N


def _stage_a(fushed, adj, centroids, W1, b1r, w2r, b2r, bnw, bnb):
    grid = (N // NB_A,)
    return pl.pallas_call(
        _stage_a_body,
        grid=grid,
        in_specs=[
            pl.BlockSpec((B, NB_A, D), lambda i: (0, i, 0)),
            pl.BlockSpec((B, NB_A, K), lambda i: (0, i, 0)),
            pl.BlockSpec((C, D), lambda i: (0, 0)),
            pl.BlockSpec((2 * D, D), lambda i: (0, 0)),
            pl.BlockSpec((1, D), lambda i: (0, 0)),
            pl.BlockSpec((1, D), lambda i: (0, 0)),
            pl.BlockSpec((1, 1), lambda i: (0, 0)),
            pl.BlockSpec((NB_A, 1), lambda i: (i, 0)),
            pl.BlockSpec((NB_A, 1), lambda i: (i, 0)),
        ],
        out_specs=[
            pl.BlockSpec((B, NB_A, 16), lambda i: (0, i, 0)),
            pl.BlockSpec((B, NB_A, K), lambda i: (0, i, 0)),
        ],
        out_shape=[
            jax.ShapeDtypeStruct((B, N, 16), jnp.float32),
            jax.ShapeDtypeStruct((B, N, K), jnp.int32),
        ],
    )(fushed, adj, centroids, W1, b1r, w2r, b2r, bnw, bnb)


def _stage_b_body(x_hbm, s_hbm, idx_hbm, out_hbm, psum_hbm,
                  idx_v, xrows_v, srows_v, out_v, psum_v, sem):
    wid = lax.axis_index("s") * NC + lax.axis_index("c")
    zero16 = jnp.zeros((16,), jnp.float32)
    for c in range(C):
        for j in range(D // 16):
            psum_v[c, pl.ds(16 * j, 16)] = zero16

    def chunk_body(t, _):
        node_base = wid * PER_W + t * CHUNK
        pltpu.sync_copy(idx_hbm.at[pl.ds(node_base * K, CHUNK * K)], idx_v)
        pltpu.async_copy(x_hbm.at[idx_v], xrows_v, sem).wait()
        pltpu.async_copy(s_hbm.at[idx_v], srows_v, sem).wait()

        def node_body(m, _):
            accs = [[zero16 for _ in range(D // 16)] for _ in range(C)]
            for k in range(K):
                xs = [xrows_v[m * K + k, pl.ds(16 * j, 16)]
                      for j in range(D // 16)]
                srow = srows_v[m * K + k, :]
                for c in range(C):
                    s = srow[c]
                    for j in range(D // 16):
                        accs[c][j] = accs[c][j] + s * xs[j]
            inv_k = 1.0 / K
            for c in range(C):
                for j in range(D // 16):
                    v = accs[c][j] * inv_k
                    out_v[m, pl.ds(c * 64 + 16 * j, 16)] = v
                    psum_v[c, pl.ds(16 * j, 16)] = (
                        psum_v[c, pl.ds(16 * j, 16)] + v)
            return ()

        lax.fori_loop(0, CHUNK, node_body, ())
        pltpu.sync_copy(out_v, out_hbm.at[pl.ds(node_base, CHUNK)])
        return ()

    lax.fori_loop(0, NCHUNK, chunk_body, ())
    pltpu.sync_copy(psum_v, psum_hbm.at[wid])


def _stage_b(x_rows, s_rows, idx2):
    mesh = plsc.VectorSubcoreMesh(core_axis_name="c", subcore_axis_name="s")
    run = pl.kernel(
        _stage_b_body,
        out_type=[
            jax.ShapeDtypeStruct((TOT, C * D), jnp.float32),
            jax.ShapeDtypeStruct((NW, C, D), jnp.float32),
        ],
        mesh=mesh,
        compiler_params=pltpu.CompilerParams(use_tc_tiling_on_sc=False),
        scratch_types=[
            pltpu.VMEM((CHUNK * K,), jnp.int32),
            pltpu.VMEM((CHUNK * K, D), jnp.float32),
            pltpu.VMEM((CHUNK * K, 16), jnp.float32),
            pltpu.VMEM((CHUNK, C * D), jnp.float32),
            pltpu.VMEM((C, D), jnp.float32),
            pltpu.SemaphoreType.DMA,
        ],
    )
    return run(x_rows, s_rows, idx2)


def _stage_c_body(psum_ref, cent_ref, out_ref):
    acc = psum_ref[0]
    for w in range(1, NW):
        acc = acc + psum_ref[w]
    u = acc * (1.0 / TOT)
    nc = (1.0 - UPDATE_RATE) * cent_ref[...] + UPDATE_RATE * u

    adj = jnp.mean(nc, axis=0, keepdims=True)
    xc = nc - adj
    nsq = jnp.sum(xc * xc, axis=1, keepdims=True)
    ones = jnp.ones_like(nsq)
    x1_ = jnp.concatenate([-2.0 * xc, nsq, ones], axis=1)
    x2_ = jnp.concatenate([xc, ones, nsq], axis=1)
    res = lax.dot_general(x1_, x2_, (((1,), (1,)), ((), ())),
                          preferred_element_type=jnp.float32)
    dist = jnp.sqrt(jnp.clip(res, 1e-30, None))
    ii = lax.broadcasted_iota(jnp.int32, (C, C), 0)
    jj = lax.broadcasted_iota(jnp.int32, (C, C), 1)
    target = jnp.where(ii == jj, 0.0, MARGIN)
    l = jnp.maximum(target - dist, 0.0)
    out_ref[...] = jnp.reshape(jnp.sum(l * l), (1, 1))


def _stage_c(psum, centroids):
    return pl.pallas_call(
        _stage_c_body,
        out_shape=jax.ShapeDtypeStruct((1, 1), jnp.float32),
    )(psum, centroids)


@jax.jit
def kernel(fushed_features, input_data, adj_mx_topk_index, centroids,
           W1, b1, W2, b2, bn_weight, bn_bias):
    b1r = jnp.reshape(b1, (1, D))
    w2r = jnp.reshape(W2, (1, D))
    b2r = jnp.reshape(b2, (1, 1))
    bnw = jnp.reshape(bn_weight, (N, 1))
    bnb = jnp.reshape(bn_bias, (N, 1))

    simi_pad, idx2 = _stage_a(fushed_features, adj_mx_topk_index, centroids,
                              W1, b1r, w2r, b2r, bnw, bnb)

    x_rows = jnp.reshape(input_data, (TOT, D))
    s_rows = jnp.reshape(simi_pad, (TOT, 16))
    idx_rows = jnp.reshape(idx2, (TOT * K,))
    out_rows, psum = _stage_b(x_rows, s_rows, idx_rows)

    loss = _stage_c(psum, centroids)
    updated_input = jnp.reshape(out_rows, (B, N, C, D))
    return updated_input, jnp.reshape(loss, ())


# speedup vs baseline: 13.1030x; 1.1151x over previous
import jax
import jax.numpy as jnp
from jax import lax
from jax.experimental import pallas as pl
from jax.experimental.pallas import tpu as pltpu
from jax.experimental.pallas import tpu_sc as plsc

B, N, K, C, D = 2, 10000, 16, 8, 64
UPDATE_RATE = 0.01
MARGIN = 0.5

NC, NS = 2, 16
NW = NC * NS
TOT = B * N
CHUNK = 8
NCH = TOT // CHUNK
QPW = -(-NCH // NW)
QMAX = (QPW + 1) // 2

NB_A = 1000


def _stage_a_body(f_ref, x_ref, cent_ref, w1_ref, b1_ref, w2_ref, b2_ref,
                  bnw_ref, bnb_ref, z_ref):
    f0 = f_ref[0]
    f1 = f_ref[1]
    inv_bd = 1.0 / (B * D)
    mean = (jnp.sum(f0, axis=1, keepdims=True)
            + jnp.sum(f1, axis=1, keepdims=True)) * inv_bd
    d0 = f0 - mean
    d1 = f1 - mean
    var = (jnp.sum(d0 * d0, axis=1, keepdims=True)
           + jnp.sum(d1 * d1, axis=1, keepdims=True)) * inv_bd
    scale = bnw_ref[...] * lax.rsqrt(var + 1e-5)
    bias = bnb_ref[...]

    w1a = w1_ref[0:D, :]
    w1b = w1_ref[D:2 * D, :]
    cpart = jnp.dot(cent_ref[...], w1b,
                    preferred_element_type=jnp.float32) + b1_ref[...]
    w2row = w2_ref[...]
    b2 = b2_ref[...]

    for b, db in ((0, d0), (1, d1)):
        ffn = db * scale + bias
        xp = jnp.dot(ffn, w1a, preferred_element_type=jnp.float32)
        cols = []
        for c in range(C):
            h = jnp.maximum(xp + cpart[c:c + 1, :], 0.0)
            sc = jnp.sum(h * w2row, axis=1, keepdims=True)
            cols.append(jnp.maximum(sc + b2, 0.0))
        s = jnp.concatenate(cols, axis=1)
        m = jnp.max(s, axis=1, keepdims=True)
        e = jnp.exp(s - m)
        simi = e / jnp.sum(e, axis=1, keepdims=True)
        pad = jnp.zeros((simi.shape[0], 128 - D - C), jnp.float32)
        z_ref[b] = jnp.concatenate([x_ref[b], simi, pad], axis=1)


def _stage_a(fushed, xinp, centroids, W1, b1r, w2r, b2r, bnw, bnb):
    grid = (N // NB_A,)
    return pl.pallas_call(
        _stage_a_body,
        grid=grid,
        in_specs=[
            pl.BlockSpec((B, NB_A, D), lambda i: (0, i, 0)),
            pl.BlockSpec((B, NB_A, D), lambda i: (0, i, 0)),
            pl.BlockSpec((C, D), lambda i: (0, 0)),
            pl.BlockSpec((2 * D, D), lambda i: (0, 0)),
            pl.BlockSpec((1, D), lambda i: (0, 0)),
            pl.BlockSpec((1, D), lambda i: (0, 0)),
            pl.BlockSpec((1, 1), lambda i: (0, 0)),
            pl.BlockSpec((NB_A, 1), lambda i: (i, 0)),
            pl.BlockSpec((NB_A, 1), lambda i: (i, 0)),
        ],
        out_specs=pl.BlockSpec((B, NB_A, 128), lambda i: (0, i, 0)),
        out_shape=jax.ShapeDtypeStruct((B, N, 128), jnp.float32),
    )(fushed, xinp, centroids, W1, b1r, w2r, b2r, bnw, bnb)


def _stage_b_body(z_hbm, idx_hbm, out_hbm, psum_hbm,
                  idx0, idx1, zr0, zr1, out_v, psum_v,
                  gsem0, gsem1):
    wid = lax.axis_index("s") * NC + lax.axis_index("c")
    idx_v = (idx0, idx1)
    zr = (zr0, zr1)
    gsem = (gsem0, gsem1)
    zero16 = jnp.zeros((16,), jnp.float32)
    for c in range(C):
        for j in range(D // 16):
            psum_v[c, pl.ds(16 * j, 16)] = zero16

    def issue(q, p):
        cid = q * NW + wid

        @pl.when(cid < NCH)
        def _():
            pltpu.sync_copy(idx_hbm.at[pl.ds(cid * CHUNK * K, CHUNK * K)],
                            idx_v[p])
            pltpu.async_copy(z_hbm.at[idx_v[p]], zr[p], gsem[p])

    issue(0, 0)
    issue(1, 1)

    def pair_body(tt, _):
        for p in (0, 1):
            q = 2 * tt + p
            cid = q * NW + wid

            @pl.when(cid < NCH)
            def _():
                pltpu.make_async_copy(z_hbm.at[idx_v[p]], zr[p],
                                      gsem[p]).wait()

                def node_body(m, _):
                    accs = [[zero16 for _ in range(D // 16)]
                            for _ in range(C)]
                    for k in range(K):
                        row = m * K + k
                        xs = [zr[p][row, pl.ds(16 * j, 16)]
                              for j in range(D // 16)]
                        srow = zr[p][row, pl.ds(D, 16)]
                        for c in range(C):
                            s = srow[c]
                            for j in range(D // 16):
                                accs[c][j] = accs[c][j] + s * xs[j]
                    inv_k = 1.0 / K
                    for c in range(C):
                        for j in range(D // 16):
                            v = accs[c][j] * inv_k
                            out_v[m * 4 + c // 2,
                                  pl.ds((c % 2) * 64 + 16 * j, 16)] = v
                            psum_v[c, pl.ds(16 * j, 16)] = (
                                psum_v[c, pl.ds(16 * j, 16)] + v)
                    return ()

                lax.fori_loop(0, CHUNK, node_body, ())
                pltpu.sync_copy(out_v,
                                out_hbm.at[pl.ds(cid * CHUNK * 4, CHUNK * 4)])

            issue(q + 2, p)
        return ()

    lax.fori_loop(0, QMAX, pair_body, ())
    pltpu.sync_copy(psum_v, psum_hbm.at[pl.ds(wid * C, C)])


def _stage_b(z_rows, idx_flat):
    mesh = plsc.VectorSubcoreMesh(core_axis_name="c", subcore_axis_name="s")
    run = pl.kernel(
        _stage_b_body,
        out_type=[
            jax.ShapeDtypeStruct((TOT * 4, 128), jnp.float32),
            jax.ShapeDtypeStruct((NW * C, D), jnp.float32),
        ],
        mesh=mesh,
        scratch_types=[
            pltpu.VMEM((CHUNK * K,), jnp.int32),
            pltpu.VMEM((CHUNK * K,), jnp.int32),
            pltpu.VMEM((CHUNK * K, 128), jnp.float32),
            pltpu.VMEM((CHUNK * K, 128), jnp.float32),
            pltpu.VMEM((CHUNK * 4, 128), jnp.float32),
            pltpu.VMEM((C, D), jnp.float32),
            pltpu.SemaphoreType.DMA,
            pltpu.SemaphoreType.DMA,
        ],
    )
    return run(z_rows, idx_flat)


def _stage_c_body(psum_ref, cent_ref, out_ref):
    acc = psum_ref[pl.ds(0, C), :]
    for w in range(1, NW):
        acc = acc + psum_ref[pl.ds(w * C, C), :]
    u = acc * (1.0 / TOT)
    nc = (1.0 - UPDATE_RATE) * cent_ref[...] + UPDATE_RATE * u

    adj = jnp.mean(nc, axis=0, keepdims=True)
    xc = nc - adj
    nsq = jnp.sum(xc * xc, axis=1, keepdims=True)
    ones = jnp.ones_like(nsq)
    x1_ = jnp.concatenate([-2.0 * xc, nsq, ones], axis=1)
    x2_ = jnp.concatenate([xc, ones, nsq], axis=1)
    res = lax.dot_general(x1_, x2_, (((1,), (1,)), ((), ())),
                          preferred_element_type=jnp.float32)
    dist = jnp.sqrt(jnp.clip(res, 1e-30, None))
    ii = lax.broadcasted_iota(jnp.int32, (C, C), 0)
    jj = lax.broadcasted_iota(jnp.int32, (C, C), 1)
    target = jnp.where(ii == jj, 0.0, MARGIN)
    l = jnp.maximum(target - dist, 0.0)
    out_ref[...] = jnp.reshape(jnp.sum(l * l), (1, 1))


def _stage_c(psum, centroids):
    return pl.pallas_call(
        _stage_c_body,
        out_shape=jax.ShapeDtypeStruct((1, 1), jnp.float32),
    )(psum, centroids)


@jax.jit
def kernel(fushed_features, input_data, adj_mx_topk_index, centroids,
           W1, b1, W2, b2, bn_weight, bn_bias):
    b1r = jnp.reshape(b1, (1, D))
    w2r = jnp.reshape(W2, (1, D))
    b2r = jnp.reshape(b2, (1, 1))
    bnw = jnp.reshape(bn_weight, (N, 1))
    bnb = jnp.reshape(bn_bias, (N, 1))
    xinp = jnp.reshape(input_data, (B, N, D))

    z = _stage_a(fushed_features, xinp, centroids,
                 W1, b1r, w2r, b2r, bnw, bnb)

    idx_flat = jnp.reshape(
        adj_mx_topk_index
        + (jnp.arange(B, dtype=jnp.int32) * N)[:, None, None],
        (TOT * K,))

    z_rows = jnp.reshape(z, (TOT, 128))
    out_rows, psum = _stage_b(z_rows, idx_flat)

    loss = _stage_c(psum, centroids)
    updated_input = jnp.reshape(out_rows, (B, N, C, D))
    return updated_input, jnp.reshape(loss, ())


# speedup vs baseline: 15.1800x; 1.1585x over previous
import jax
import jax.numpy as jnp
from jax import lax
from jax.experimental import pallas as pl
from jax.experimental.pallas import tpu as pltpu
from jax.experimental.pallas import tpu_sc as plsc

B, N, K, C, D = 2, 10000, 16, 8, 64
UPDATE_RATE = 0.01
MARGIN = 0.5

NC, NS = 2, 16
NW = NC * NS
TOT = B * N
CHUNK = 8
NCH = TOT // CHUNK
NCH_B = N // CHUNK
QPW = -(-NCH // NW)
QMAX = (QPW + 1) // 2

NB_A = 2000


def _stage_a_body(f_ref, x_ref, cent_ref, w1_ref, b1_ref, w2_ref, b2_ref,
                  bn_ref, z_ref):
    f0 = f_ref[0]
    f1 = f_ref[1]
    inv_bd = 1.0 / (B * D)
    mean = (jnp.sum(f0, axis=1, keepdims=True)
            + jnp.sum(f1, axis=1, keepdims=True)) * inv_bd
    d0 = f0 - mean
    d1 = f1 - mean
    var = (jnp.sum(d0 * d0, axis=1, keepdims=True)
           + jnp.sum(d1 * d1, axis=1, keepdims=True)) * inv_bd
    scale = bn_ref[:, 0:1] * lax.rsqrt(var + 1e-5)
    bias = bn_ref[:, 1:2]

    w1a = w1_ref[0:D, :]
    w1b = w1_ref[D:2 * D, :]
    cpart = jnp.dot(cent_ref[...], w1b,
                    preferred_element_type=jnp.float32) + b1_ref[...]
    cp8 = jnp.concatenate([cpart[c:c + 1, :] for c in range(C)], axis=1)
    w2rep = jnp.concatenate([w2_ref[...]] * C, axis=0)
    rr = lax.broadcasted_iota(jnp.int32, (C * D, C), 0)
    cc = lax.broadcasted_iota(jnp.int32, (C * D, C), 1)
    w2blk = jnp.where(rr // D == cc, w2rep, 0.0)
    b2 = b2_ref[...]

    for b, db in ((0, d0), (1, d1)):
        ffn = db * scale + bias
        xp = jnp.dot(ffn, w1a, preferred_element_type=jnp.float32)
        xp8 = jnp.concatenate([xp] * C, axis=1)
        h = jnp.maximum(xp8 + cp8, 0.0)
        sg = jnp.dot(h, w2blk, preferred_element_type=jnp.float32)
        s = jnp.maximum(sg + b2, 0.0)
        m = jnp.max(s, axis=1, keepdims=True)
        e = jnp.exp(s - m)
        simi = e / jnp.sum(e, axis=1, keepdims=True)
        pad = jnp.zeros((simi.shape[0], 128 - D - C), jnp.float32)
        z_ref[b] = jnp.concatenate([x_ref[b, 0], simi, pad], axis=1)


def _stage_a(fushed, input_data, centroids, W1, b1r, W2, b2r, bnpack):
    grid = (N // NB_A,)
    return pl.pallas_call(
        _stage_a_body,
        grid=grid,
        in_specs=[
            pl.BlockSpec((B, NB_A, D), lambda i: (0, i, 0)),
            pl.BlockSpec((B, 1, NB_A, D), lambda i: (0, 0, i, 0)),
            pl.BlockSpec((C, D), lambda i: (0, 0)),
            pl.BlockSpec((2 * D, D), lambda i: (0, 0)),
            pl.BlockSpec((1, D), lambda i: (0, 0)),
            pl.BlockSpec((D, 1), lambda i: (0, 0)),
            pl.BlockSpec((1, 1), lambda i: (0, 0)),
            pl.BlockSpec((NB_A, 2), lambda i: (i, 0)),
        ],
        out_specs=pl.BlockSpec((B, NB_A, 128), lambda i: (0, i, 0)),
        out_shape=jax.ShapeDtypeStruct((B, N, 128), jnp.float32),
    )(fushed, input_data, centroids, W1, b1r, W2, b2r, bnpack)


def _stage_b_body(z_hbm, idx_hbm, out_hbm, psum_hbm,
                  idx0, idx1, zr0, zr1, out_v, psum_v,
                  gsem0, gsem1):
    wid = lax.axis_index("s") * NC + lax.axis_index("c")
    idx_v = (idx0, idx1)
    zr = (zr0, zr1)
    gsem = (gsem0, gsem1)
    zero16 = jnp.zeros((16,), jnp.float32)
    for c in range(C):
        for j in range(D // 16):
            psum_v[c, pl.ds(16 * j, 16)] = zero16

    def issue(q, p):
        cid = q * NW + wid

        @pl.when(cid < NCH)
        def _():
            pltpu.sync_copy(idx_hbm.at[pl.ds(cid * CHUNK * K, CHUNK * K)],
                            idx_v[p])
            pltpu.async_copy(z_hbm.at[idx_v[p]], zr[p], gsem[p])

    issue(0, 0)
    issue(1, 1)

    def pair_body(tt, _):
        for p in (0, 1):
            q = 2 * tt + p
            cid = q * NW + wid

            @pl.when(cid < NCH)
            def _():
                pltpu.make_async_copy(z_hbm.at[idx_v[p]], zr[p],
                                      gsem[p]).wait()

                def node_body(m, _):
                    accs = [[zero16 for _ in range(D // 16)]
                            for _ in range(C)]
                    for k in range(K):
                        row = m * K + k
                        xs = [zr[p][row, pl.ds(16 * j, 16)]
                              for j in range(D // 16)]
                        srow = zr[p][row, pl.ds(D, 16)]
                        for c in range(C):
                            s = srow[c]
                            for j in range(D // 16):
                                accs[c][j] = accs[c][j] + s * xs[j]
                    inv_k = 1.0 / K
                    for c in range(C):
                        for j in range(D // 16):
                            v = accs[c][j] * inv_k
                            out_v[m, c, pl.ds(16 * j, 16)] = v
                            psum_v[c, pl.ds(16 * j, 16)] = (
                                psum_v[c, pl.ds(16 * j, 16)] + v)
                    return ()

                lax.fori_loop(0, CHUNK, node_body, ())
                bq = cid // NCH_B
                n0 = (cid - bq * NCH_B) * CHUNK
                pltpu.sync_copy(out_v, out_hbm.at[bq, pl.ds(n0, CHUNK)])

            issue(q + 2, p)
        return ()

    lax.fori_loop(0, QMAX, pair_body, ())
    pltpu.sync_copy(psum_v, psum_hbm.at[wid])


def _stage_b(z_rows, idx_flat):
    mesh = plsc.VectorSubcoreMesh(core_axis_name="c", subcore_axis_name="s")
    run = pl.kernel(
        _stage_b_body,
        out_type=[
            jax.ShapeDtypeStruct((B, N, C, D), jnp.float32),
            jax.ShapeDtypeStruct((NW, C, D), jnp.float32),
        ],
        mesh=mesh,
        scratch_types=[
            pltpu.VMEM((CHUNK * K,), jnp.int32),
            pltpu.VMEM((CHUNK * K,), jnp.int32),
            pltpu.VMEM((CHUNK * K, 128), jnp.float32),
            pltpu.VMEM((CHUNK * K, 128), jnp.float32),
            pltpu.VMEM((CHUNK, C, D), jnp.float32),
            pltpu.VMEM((C, D), jnp.float32),
            pltpu.SemaphoreType.DMA,
            pltpu.SemaphoreType.DMA,
        ],
    )
    return run(z_rows, idx_flat)


def _stage_c_body(psum_ref, cent_ref, out_ref):
    acc = psum_ref[0]
    for w in range(1, NW):
        acc = acc + psum_ref[w]
    u = acc * (1.0 / TOT)
    nc = (1.0 - UPDATE_RATE) * cent_ref[...] + UPDATE_RATE * u

    adj = jnp.mean(nc, axis=0, keepdims=True)
    xc = nc - adj
    nsq = jnp.sum(xc * xc, axis=1, keepdims=True)
    ones = jnp.ones_like(nsq)
    x1_ = jnp.concatenate([-2.0 * xc, nsq, ones], axis=1)
    x2_ = jnp.concatenate([xc, ones, nsq], axis=1)
    res = lax.dot_general(x1_, x2_, (((1,), (1,)), ((), ())),
                          preferred_element_type=jnp.float32)
    dist = jnp.sqrt(jnp.clip(res, 1e-30, None))
    ii = lax.broadcasted_iota(jnp.int32, (C, C), 0)
    jj = lax.broadcasted_iota(jnp.int32, (C, C), 1)
    target = jnp.where(ii == jj, 0.0, MARGIN)
    l = jnp.maximum(target - dist, 0.0)
    out_ref[...] = jnp.reshape(jnp.sum(l * l), (1, 1))


def _stage_c(psum, centroids):
    return pl.pallas_call(
        _stage_c_body,
        out_shape=jax.ShapeDtypeStruct((1, 1), jnp.float32),
    )(psum, centroids)


@jax.jit
def kernel(fushed_features, input_data, adj_mx_topk_index, centroids,
           W1, b1, W2, b2, bn_weight, bn_bias):
    b1r = jnp.reshape(b1, (1, D))
    b2r = jnp.reshape(b2, (1, 1))
    bnpack = jnp.stack([bn_weight, bn_bias], axis=1)

    z = _stage_a(fushed_features, input_data, centroids,
                 W1, b1r, W2, b2r, bnpack)

    idx_flat = jnp.reshape(
        adj_mx_topk_index
        + (jnp.arange(B, dtype=jnp.int32) * N)[:, None, None],
        (TOT * K,))

    z_rows = jnp.reshape(z, (TOT, 128))
    updated_input, psum = _stage_b(z_rows, idx_flat)

    loss = _stage_c(psum, centroids)
    return updated_input, jnp.reshape(loss, ())


# speedup vs baseline: 19.7245x; 1.2994x over previous
import jax
import jax.numpy as jnp
from jax import lax
from jax.experimental import pallas as pl
from jax.experimental.pallas import tpu as pltpu
from jax.experimental.pallas import tpu_sc as plsc

B, N, K, C, D = 2, 10000, 16, 8, 64
UPDATE_RATE = 0.01
MARGIN = 0.5

NC, NS = 2, 16
NW = NC * NS
TOT = B * N
CHUNK = 8
NCH = TOT // CHUNK
NCH_B = N // CHUNK
QPW = -(-NCH // NW)
QMAX = (QPW + 1) // 2

NB_A = 2000


def _stage_a_body(f_ref, x_ref, cent_ref, w1_ref, b1_ref, w2_ref, b2_ref,
                  bn_ref, z_ref):
    f0 = f_ref[0]
    f1 = f_ref[1]
    inv_bd = 1.0 / (B * D)
    mean = (jnp.sum(f0, axis=1, keepdims=True)
            + jnp.sum(f1, axis=1, keepdims=True)) * inv_bd
    d0 = f0 - mean
    d1 = f1 - mean
    var = (jnp.sum(d0 * d0, axis=1, keepdims=True)
           + jnp.sum(d1 * d1, axis=1, keepdims=True)) * inv_bd
    scale = bn_ref[:, 0:1] * lax.rsqrt(var + 1e-5)
    bias = bn_ref[:, 1:2]

    w1a = w1_ref[0:D, :]
    w1b = w1_ref[D:2 * D, :]
    cpart = jnp.dot(cent_ref[...], w1b,
                    preferred_element_type=jnp.float32) + b1_ref[...]
    cp8 = jnp.concatenate([cpart[c:c + 1, :] for c in range(C)], axis=1)
    w2rep = jnp.concatenate([w2_ref[...]] * C, axis=0)
    rr = lax.broadcasted_iota(jnp.int32, (C * D, C), 0)
    cc = lax.broadcasted_iota(jnp.int32, (C * D, C), 1)
    w2blk = jnp.where(rr // D == cc, w2rep, 0.0)
    b2 = b2_ref[...]

    for b, db in ((0, d0), (1, d1)):
        ffn = db * scale + bias
        xp = jnp.dot(ffn, w1a, preferred_element_type=jnp.float32)
        xp8 = jnp.concatenate([xp] * C, axis=1)
        h = jnp.maximum(xp8 + cp8, 0.0)
        sg = jnp.dot(h, w2blk, preferred_element_type=jnp.float32)
        s = jnp.maximum(sg + b2, 0.0)
        m = jnp.max(s, axis=1, keepdims=True)
        e = jnp.exp(s - m)
        simi = e / jnp.sum(e, axis=1, keepdims=True)
        sb = lax.bitcast_convert_type(
            (simi * (1.0 / K)).astype(jnp.bfloat16), jnp.uint16)
        sw = sb.astype(jnp.uint32)
        sw = sw | (sw << 16)
        spk = lax.bitcast_convert_type(sw, jnp.float32)
        pad = jnp.zeros((simi.shape[0], 128 - 32 - C), jnp.float32)
        z_ref[b] = jnp.concatenate([x_ref[b], spk, pad], axis=1)


def _stage_a(fushed, xpk, centroids, W1, b1r, W2, b2r, bnpack):
    grid = (N // NB_A,)
    return pl.pallas_call(
        _stage_a_body,
        grid=grid,
        in_specs=[
            pl.BlockSpec((B, NB_A, D), lambda i: (0, i, 0)),
            pl.BlockSpec((B, NB_A, 32), lambda i: (0, i, 0)),
            pl.BlockSpec((C, D), lambda i: (0, 0)),
            pl.BlockSpec((2 * D, D), lambda i: (0, 0)),
            pl.BlockSpec((1, D), lambda i: (0, 0)),
            pl.BlockSpec((D, 1), lambda i: (0, 0)),
            pl.BlockSpec((1, 1), lambda i: (0, 0)),
            pl.BlockSpec((NB_A, 2), lambda i: (i, 0)),
        ],
        out_specs=pl.BlockSpec((B, NB_A, 128), lambda i: (0, i, 0)),
        out_shape=jax.ShapeDtypeStruct((B, N, 128), jnp.float32),
    )(fushed, xpk, centroids, W1, b1r, W2, b2r, bnpack)


def _stage_b_body(z_hbm, idx_hbm, out_hbm, psum_hbm,
                  idx0, idx1, zr0, zr1, out_v, psum_v, psf_v,
                  gsem0, gsem1):
    wid = lax.axis_index("s") * NC + lax.axis_index("c")
    idx_v = (idx0, idx1)
    zr = (zr0, zr1)
    gsem = (gsem0, gsem1)
    zero_bf = jnp.zeros((32,), jnp.bfloat16)
    zero16 = jnp.zeros((16,), jnp.float32)
    for c in range(C):
        for h in range(2):
            psum_v[c, pl.ds(16 * h, 16)] = zero16

    def issue(q, p):
        cid = q * NW + wid

        @pl.when(cid < NCH)
        def _():
            pltpu.sync_copy(idx_hbm.at[pl.ds(cid * CHUNK * K, CHUNK * K)],
                            idx_v[p])
            pltpu.async_copy(z_hbm.at[idx_v[p]], zr[p], gsem[p])

    issue(0, 0)
    issue(1, 1)

    def pair_body(tt, _):
        for p in (0, 1):
            q = 2 * tt + p
            cid = q * NW + wid

            @pl.when(cid < NCH)
            def _():
                pltpu.make_async_copy(z_hbm.at[idx_v[p]], zr[p],
                                      gsem[p]).wait()

                def node_body(m, _):
                    accs = [[zero_bf, zero_bf] for _ in range(C)]
                    for k in range(K):
                        row = m * K + k
                        xh = [plsc.bitcast(zr[p][row, pl.ds(16 * h, 16)],
                                           jnp.bfloat16) for h in range(2)]
                        srow = zr[p][row, pl.ds(32, 16)]
                        for c in range(C):
                            sb = plsc.bitcast(
                                lax.broadcast_in_dim(srow[c], (16,), ()),
                                jnp.bfloat16)
                            accs[c][0] = accs[c][0] + sb * xh[0]
                            accs[c][1] = accs[c][1] + sb * xh[1]
                    for c in range(C):
                        for h in range(2):
                            a, b = plsc.unpack(
                                accs[c][h],
                                format=plsc.PackFormat.INTERLEAVED)
                            out_v[m, c, pl.ds(32 * h, 16)] = a
                            out_v[m, c, pl.ds(32 * h + 16, 16)] = b
                            pb = plsc.bitcast(
                                psum_v[c, pl.ds(16 * h, 16)],
                                jnp.bfloat16) + accs[c][h]
                            psum_v[c, pl.ds(16 * h, 16)] = plsc.bitcast(
                                pb, jnp.float32)
                    return ()

                lax.fori_loop(0, CHUNK, node_body, ())
                bq = cid // NCH_B
                n0 = (cid - bq * NCH_B) * CHUNK
                pltpu.sync_copy(out_v, out_hbm.at[bq, pl.ds(n0, CHUNK)])

            issue(q + 2, p)
        return ()

    lax.fori_loop(0, QMAX, pair_body, ())
    for c in range(C):
        for h in range(2):
            a, b = plsc.unpack(
                plsc.bitcast(psum_v[c, pl.ds(16 * h, 16)], jnp.bfloat16),
                format=plsc.PackFormat.INTERLEAVED)
            psf_v[c, pl.ds(32 * h, 16)] = a
            psf_v[c, pl.ds(32 * h + 16, 16)] = b
    pltpu.sync_copy(psf_v, psum_hbm.at[wid])


def _stage_b(z_rows, idx_flat):
    mesh = plsc.VectorSubcoreMesh(core_axis_name="c", subcore_axis_name="s")
    run = pl.kernel(
        _stage_b_body,
        out_type=[
            jax.ShapeDtypeStruct((B, N, C, D), jnp.float32),
            jax.ShapeDtypeStruct((NW, C, D), jnp.float32),
        ],
        mesh=mesh,
        compiler_params=pltpu.CompilerParams(needs_layout_passes=False),
        scratch_types=[
            pltpu.VMEM((CHUNK * K,), jnp.int32),
            pltpu.VMEM((CHUNK * K,), jnp.int32),
            pltpu.VMEM((CHUNK * K, 128), jnp.float32),
            pltpu.VMEM((CHUNK * K, 128), jnp.float32),
            pltpu.VMEM((CHUNK, C, D), jnp.float32),
            pltpu.VMEM((C, 32), jnp.float32),
            pltpu.VMEM((C, D), jnp.float32),
            pltpu.SemaphoreType.DMA,
            pltpu.SemaphoreType.DMA,
        ],
    )
    return run(z_rows, idx_flat)


def _stage_c_body(psum_ref, cent_ref, out_ref):
    acc = psum_ref[0]
    for w in range(1, NW):
        acc = acc + psum_ref[w]
    u = acc * (1.0 / TOT)
    nc = (1.0 - UPDATE_RATE) * cent_ref[...] + UPDATE_RATE * u

    adj = jnp.mean(nc, axis=0, keepdims=True)
    xc = nc - adj
    nsq = jnp.sum(xc * xc, axis=1, keepdims=True)
    ones = jnp.ones_like(nsq)
    x1_ = jnp.concatenate([-2.0 * xc, nsq, ones], axis=1)
    x2_ = jnp.concatenate([xc, ones, nsq], axis=1)
    res = lax.dot_general(x1_, x2_, (((1,), (1,)), ((), ())),
                          preferred_element_type=jnp.float32)
    dist = jnp.sqrt(jnp.clip(res, 1e-30, None))
    ii = lax.broadcasted_iota(jnp.int32, (C, C), 0)
    jj = lax.broadcasted_iota(jnp.int32, (C, C), 1)
    target = jnp.where(ii == jj, 0.0, MARGIN)
    l = jnp.maximum(target - dist, 0.0)
    out_ref[...] = jnp.reshape(jnp.sum(l * l), (1, 1))


def _stage_c(psum, centroids):
    return pl.pallas_call(
        _stage_c_body,
        out_shape=jax.ShapeDtypeStruct((1, 1), jnp.float32),
    )(psum, centroids)


@jax.jit
def kernel(fushed_features, input_data, adj_mx_topk_index, centroids,
           W1, b1, W2, b2, bn_weight, bn_bias):
    b1r = jnp.reshape(b1, (1, D))
    b2r = jnp.reshape(b2, (1, 1))
    bnpack = jnp.stack([bn_weight, bn_bias], axis=1)

    xb = jnp.reshape(input_data, (B, N, D)).astype(jnp.bfloat16)
    xu = lax.bitcast_convert_type(xb, jnp.uint16).astype(jnp.uint32)
    words = []
    for g in range(2):
        lo = xu[:, :, 32 * g:32 * g + 16]
        hi = xu[:, :, 32 * g + 16:32 * g + 32]
        words.append(lo | (hi << 16))
    xpk = lax.bitcast_convert_type(
        jnp.concatenate(words, axis=-1), jnp.float32)

    z = _stage_a(fushed_features, xpk, centroids,
                 W1, b1r, W2, b2r, bnpack)

    idx_flat = jnp.reshape(
        adj_mx_topk_index
        + (jnp.arange(B, dtype=jnp.int32) * N)[:, None, None],
        (TOT * K,))

    z_rows = jnp.reshape(z, (TOT, 128))
    updated_input, psum = _stage_b(z_rows, idx_flat)

    loss = _stage_c(psum, centroids)
    return updated_input, jnp.reshape(loss, ())


# speedup vs baseline: 24.0584x; 1.2197x over previous
import jax
import jax.numpy as jnp
from jax import lax
from jax.experimental import pallas as pl
from jax.experimental.pallas import tpu as pltpu
from jax.experimental.pallas import tpu_sc as plsc

B, N, K, C, D = 2, 10000, 16, 8, 64
UPDATE_RATE = 0.01
MARGIN = 0.5

NC, NS = 2, 16
NW = NC * NS
TOT = B * N
CHUNK = 8
NCH = TOT // CHUNK
NCH_B = N // CHUNK
QPW = -(-NCH // NW)
QMAX = (QPW + 1) // 2

NB_A = 2000


def _stage_a_body(f_ref, x_ref, cent_ref, w1_ref, b1_ref, w2_ref, b2_ref,
                  bn_ref, z_ref):
    f0 = f_ref[0]
    f1 = f_ref[1]
    inv_bd = 1.0 / (B * D)
    mean = (jnp.sum(f0, axis=1, keepdims=True)
            + jnp.sum(f1, axis=1, keepdims=True)) * inv_bd
    d0 = f0 - mean
    d1 = f1 - mean
    var = (jnp.sum(d0 * d0, axis=1, keepdims=True)
           + jnp.sum(d1 * d1, axis=1, keepdims=True)) * inv_bd
    scale = bn_ref[:, 0:1] * lax.rsqrt(var + 1e-5)
    bias = bn_ref[:, 1:2]

    w1a = w1_ref[0:D, :]
    w1b = w1_ref[D:2 * D, :]
    cpart = jnp.dot(cent_ref[...], w1b,
                    preferred_element_type=jnp.float32) + b1_ref[...]
    cp8 = jnp.concatenate([cpart[c:c + 1, :] for c in range(C)], axis=1)
    w2rep = jnp.concatenate([w2_ref[...]] * C, axis=0)
    rr = lax.broadcasted_iota(jnp.int32, (C * D, C), 0)
    cc = lax.broadcasted_iota(jnp.int32, (C * D, C), 1)
    w2blk = jnp.where(rr // D == cc, w2rep, 0.0)
    b2 = b2_ref[...]

    for b, db in ((0, d0), (1, d1)):
        ffn = db * scale + bias
        xp = jnp.dot(ffn, w1a, preferred_element_type=jnp.float32)
        xp8 = jnp.concatenate([xp] * C, axis=1)
        h = jnp.maximum(xp8 + cp8, 0.0)
        sg = jnp.dot(h, w2blk, preferred_element_type=jnp.float32)
        s = jnp.maximum(sg + b2, 0.0)
        m = jnp.max(s, axis=1, keepdims=True)
        e = jnp.exp(s - m)
        simi = e / jnp.sum(e, axis=1, keepdims=True)
        sb = lax.bitcast_convert_type(
            (simi * (1.0 / K)).astype(jnp.bfloat16), jnp.uint16)
        sw = sb.astype(jnp.uint32)
        sw = sw | (sw << 16)
        spk = lax.bitcast_convert_type(sw, jnp.float32)
        pad = jnp.zeros((simi.shape[0], 128 - 32 - C), jnp.float32)
        z_ref[b] = jnp.concatenate([x_ref[b], spk, pad], axis=1)


def _stage_a(fushed, xpk, centroids, W1, b1r, W2, b2r, bnpack):
    grid = (N // NB_A,)
    return pl.pallas_call(
        _stage_a_body,
        grid=grid,
        in_specs=[
            pl.BlockSpec((B, NB_A, D), lambda i: (0, i, 0)),
            pl.BlockSpec((B, NB_A, 32), lambda i: (0, i, 0)),
            pl.BlockSpec((C, D), lambda i: (0, 0)),
            pl.BlockSpec((2 * D, D), lambda i: (0, 0)),
            pl.BlockSpec((1, D), lambda i: (0, 0)),
            pl.BlockSpec((D, 1), lambda i: (0, 0)),
            pl.BlockSpec((1, 1), lambda i: (0, 0)),
            pl.BlockSpec((NB_A, 2), lambda i: (i, 0)),
        ],
        out_specs=pl.BlockSpec((B, NB_A, 128), lambda i: (0, i, 0)),
        out_shape=jax.ShapeDtypeStruct((B, N, 128), jnp.float32),
    )(fushed, xpk, centroids, W1, b1r, W2, b2r, bnpack)


def _stage_b_body(z_hbm, idx_hbm, out_hbm, psum_hbm,
                  idx0, idx1, zr0, zr1, out0, out1, psum_v, psf_v,
                  gsem0, gsem1, isem0, isem1, osem0, osem1):
    wid = lax.axis_index("s") * NC + lax.axis_index("c")
    idx_v = (idx0, idx1)
    zr = (zr0, zr1)
    out_v = (out0, out1)
    gsem = (gsem0, gsem1)
    isem = (isem0, isem1)
    osem = (osem0, osem1)
    zero_bf = jnp.zeros((32,), jnp.bfloat16)
    zero16 = jnp.zeros((16,), jnp.float32)
    for c in range(C):
        for h in range(2):
            psum_v[c, pl.ds(16 * h, 16)] = zero16

    def out_slice(cid):
        bq = cid // NCH_B
        n0 = (cid - bq * NCH_B) * CHUNK
        return out_hbm.at[bq, pl.ds(n0, CHUNK)]

    for p in (0, 1):
        cid0 = p * NW + wid
        pltpu.sync_copy(idx_hbm.at[pl.ds(cid0 * CHUNK * K, CHUNK * K)],
                        idx_v[p])
        pltpu.async_copy(z_hbm.at[idx_v[p]], zr[p], gsem[p])

    def pair_body(tt, _):
        for p in (0, 1):
            q = 2 * tt + p
            cid = q * NW + wid
            cid2 = cid + 2 * NW

            @pl.when(cid < NCH)
            def _():
                pltpu.make_async_copy(z_hbm.at[idx_v[p]], zr[p],
                                      gsem[p]).wait()

                @pl.when(cid2 < NCH)
                def _():
                    pltpu.async_copy(
                        idx_hbm.at[pl.ds(cid2 * CHUNK * K, CHUNK * K)],
                        idx_v[p], isem[p])

                @pl.when(cid >= 2 * NW)
                def _():
                    pltpu.make_async_copy(out_v[p], out_slice(cid),
                                          osem[p]).wait()

                def node_body(m, psums):
                    accs = [[zero_bf, zero_bf] for _ in range(C)]
                    for k in range(K):
                        row = m * K + k
                        xh = [plsc.bitcast(zr[p][row, pl.ds(16 * h, 16)],
                                           jnp.bfloat16) for h in range(2)]
                        srow = zr[p][row, pl.ds(32, 16)]
                        for c in range(C):
                            sb = plsc.bitcast(
                                lax.broadcast_in_dim(srow[c], (16,), ()),
                                jnp.bfloat16)
                            accs[c][0] = accs[c][0] + sb * xh[0]
                            accs[c][1] = accs[c][1] + sb * xh[1]
                    new_psums = []
                    for c in range(C):
                        for h in range(2):
                            a, b = plsc.unpack(
                                accs[c][h],
                                format=plsc.PackFormat.INTERLEAVED)
                            out_v[p][m, c, pl.ds(32 * h, 16)] = a
                            out_v[p][m, c, pl.ds(32 * h + 16, 16)] = b
                            new_psums.append(psums[2 * c + h] + accs[c][h])
                    return tuple(new_psums)

                psums = lax.fori_loop(0, CHUNK, node_body,
                                      (zero_bf,) * (2 * C))
                for c in range(C):
                    for h in range(2):
                        pb = plsc.bitcast(psum_v[c, pl.ds(16 * h, 16)],
                                          jnp.bfloat16) + psums[2 * c + h]
                        psum_v[c, pl.ds(16 * h, 16)] = plsc.bitcast(
                            pb, jnp.float32)
                pltpu.async_copy(out_v[p], out_slice(cid), osem[p])

                @pl.when(cid2 < NCH)
                def _():
                    pltpu.make_async_copy(
                        idx_hbm.at[pl.ds(cid2 * CHUNK * K, CHUNK * K)],
                        idx_v[p], isem[p]).wait()
                    pltpu.async_copy(z_hbm.at[idx_v[p]], zr[p], gsem[p])

        return ()

    lax.fori_loop(0, QMAX, pair_body, ())
    nq = (NCH - wid + NW - 1) // NW
    for p in (0, 1):
        qlast = nq - 2 + ((nq - p) % 2)
        pltpu.make_async_copy(out_v[p], out_slice(qlast * NW + wid),
                              osem[p]).wait()
    for c in range(C):
        for h in range(2):
            a, b = plsc.unpack(
                plsc.bitcast(psum_v[c, pl.ds(16 * h, 16)], jnp.bfloat16),
                format=plsc.PackFormat.INTERLEAVED)
            psf_v[c, pl.ds(32 * h, 16)] = a
            psf_v[c, pl.ds(32 * h + 16, 16)] = b
    pltpu.sync_copy(psf_v, psum_hbm.at[wid])


def _stage_b(z_rows, idx_flat):
    mesh = plsc.VectorSubcoreMesh(core_axis_name="c", subcore_axis_name="s")
    run = pl.kernel(
        _stage_b_body,
        out_type=[
            jax.ShapeDtypeStruct((B, N, C, D), jnp.float32),
            jax.ShapeDtypeStruct((NW, C, D), jnp.float32),
        ],
        mesh=mesh,
        compiler_params=pltpu.CompilerParams(needs_layout_passes=False),
        scratch_types=[
            pltpu.VMEM((CHUNK * K,), jnp.int32),
            pltpu.VMEM((CHUNK * K,), jnp.int32),
            pltpu.VMEM((CHUNK * K, 128), jnp.float32),
            pltpu.VMEM((CHUNK * K, 128), jnp.float32),
            pltpu.VMEM((CHUNK, C, D), jnp.float32),
            pltpu.VMEM((CHUNK, C, D), jnp.float32),
            pltpu.VMEM((C, 32), jnp.float32),
            pltpu.VMEM((C, D), jnp.float32),
            pltpu.SemaphoreType.DMA,
            pltpu.SemaphoreType.DMA,
            pltpu.SemaphoreType.DMA,
            pltpu.SemaphoreType.DMA,
            pltpu.SemaphoreType.DMA,
            pltpu.SemaphoreType.DMA,
        ],
    )
    return run(z_rows, idx_flat)


def _stage_c_body(psum_ref, cent_ref, out_ref):
    acc = psum_ref[0]
    for w in range(1, NW):
        acc = acc + psum_ref[w]
    u = acc * (1.0 / TOT)
    nc = (1.0 - UPDATE_RATE) * cent_ref[...] + UPDATE_RATE * u

    adj = jnp.mean(nc, axis=0, keepdims=True)
    xc = nc - adj
    nsq = jnp.sum(xc * xc, axis=1, keepdims=True)
    ones = jnp.ones_like(nsq)
    x1_ = jnp.concatenate([-2.0 * xc, nsq, ones], axis=1)
    x2_ = jnp.concatenate([xc, ones, nsq], axis=1)
    res = lax.dot_general(x1_, x2_, (((1,), (1,)), ((), ())),
                          preferred_element_type=jnp.float32)
    dist = jnp.sqrt(jnp.clip(res, 1e-30, None))
    ii = lax.broadcasted_iota(jnp.int32, (C, C), 0)
    jj = lax.broadcasted_iota(jnp.int32, (C, C), 1)
    target = jnp.where(ii == jj, 0.0, MARGIN)
    l = jnp.maximum(target - dist, 0.0)
    out_ref[...] = jnp.reshape(jnp.sum(l * l), (1, 1))


def _stage_c(psum, centroids):
    return pl.pallas_call(
        _stage_c_body,
        out_shape=jax.ShapeDtypeStruct((1, 1), jnp.float32),
    )(psum, centroids)


@jax.jit
def kernel(fushed_features, input_data, adj_mx_topk_index, centroids,
           W1, b1, W2, b2, bn_weight, bn_bias):
    b1r = jnp.reshape(b1, (1, D))
    b2r = jnp.reshape(b2, (1, 1))
    bnpack = jnp.stack([bn_weight, bn_bias], axis=1)

    xb = jnp.reshape(input_data, (B, N, D)).astype(jnp.bfloat16)
    xu = lax.bitcast_convert_type(xb, jnp.uint16).astype(jnp.uint32)
    words = []
    for g in range(2):
        lo = xu[:, :, 32 * g:32 * g + 16]
        hi = xu[:, :, 32 * g + 16:32 * g + 32]
        words.append(lo | (hi << 16))
    xpk = lax.bitcast_convert_type(
        jnp.concatenate(words, axis=-1), jnp.float32)

    z = _stage_a(fushed_features, xpk, centroids,
                 W1, b1r, W2, b2r, bnpack)

    idx_flat = jnp.reshape(
        adj_mx_topk_index
        + (jnp.arange(B, dtype=jnp.int32) * N)[:, None, None],
        (TOT * K,))

    z_rows = jnp.reshape(z, (TOT, 128))
    updated_input, psum = _stage_b(z_rows, idx_flat)

    loss = _stage_c(psum, centroids)
    return updated_input, jnp.reshape(loss, ())
